# baseline (speedup 1.0000x reference)
import functools

import jax
import jax.numpy as jnp
from jax import lax
from jax.experimental import pallas as pl
from jax.experimental.pallas import tpu as pltpu
from jax.experimental.pallas import tpu_sc as plsc

N = 10000
E = 160000
D = 100
DP = 128
W2 = 2 * DP
NACC = 10112
NW = 32
K = 64
EPW = 5120
EPAD = NW * EPW
NCHUNK = EPW // K
RPS = NACC // 16

_f32 = jnp.float32
_i32 = jnp.int32


def _sc_body(xcat, srcs, dsts, typs, nrms, a0h, a1h, outp,
             sb0, db0, tb0, nb0, sb1, db1, tb1, nb1,
             sb2, db2, tb2, nb2, sb3, db3, tb3, nb3,
             rows0, rows1, msg, a0v, a1v,
             acc, sem0, sem1, semi0, semi1, semi2, semi3):
    c = lax.axis_index("c")
    s = lax.axis_index("s")
    wbase = (c * 16 + s) * EPW

    pltpu.sync_copy(a0h, a0v)
    pltpu.sync_copy(a1h, a1v)

    lanes = lax.iota(_i32, 16)
    cntv = jnp.where(lanes == 4, 1.0, 0.0).astype(_f32)
    zv = jnp.zeros((16,), _f32)

    def zb(j, carry):
        for i in range(DP // 16):
            msg[j, pl.ds(i * 16, 16)] = zv
        return carry
    lax.fori_loop(0, K, zb, 0)

    nfull = RPS // K
    for t in range(nfull):
        pltpu.sync_copy(msg, acc.at[pl.ds(s * RPS + t * K, K)])
    rem = RPS - nfull * K
    if rem:
        pltpu.sync_copy(msg.at[pl.ds(0, rem)],
                        acc.at[pl.ds(s * RPS + nfull * K, rem)])
    plsc.subcore_barrier()

    def idx_copies(g, sb, db, tb, nb, semi):
        base = wbase + g * K
        return [
            pltpu.make_async_copy(srcs.at[pl.ds(base, K)], sb, semi),
            pltpu.make_async_copy(dsts.at[pl.ds(base, K)], db, semi),
            pltpu.make_async_copy(typs.at[pl.ds(base, K)], tb, semi),
            pltpu.make_async_copy(nrms.at[pl.ds(base, K)], nb, semi),
        ]

    def idx_start(g, sb, db, tb, nb, semi):
        for cp in idx_copies(g, sb, db, tb, nb, semi):
            cp.start()

    def idx_wait(g, sb, db, tb, nb, semi):
        for cp in idx_copies(g, sb, db, tb, nb, semi):
            cp.wait()

    def gather_start(sb, rows, sem):
        pltpu.make_async_copy(xcat.at[sb], rows, sem).start()

    def compute(sb, db, tb, nb, rows, sem):
        pltpu.make_async_copy(xcat.at[sb], rows, sem).wait()

        def _ebody(j, carry):
            jv = jnp.broadcast_to(j, (16,)).astype(_i32)
            tj = plsc.load_gather(tb, [jv])
            nj = plsc.load_gather(nb, [jv])
            s0 = plsc.load_gather(a0v, [tj]) * nj
            s1 = plsc.load_gather(a1v, [tj]) * nj
            r0s = [rows[j, pl.ds(i * 16, 16)] for i in range(DP // 16)]
            r1s = [rows[j, pl.ds(DP + i * 16, 16)] for i in range(DP // 16)]
            ms = [r0s[i] * s0 + r1s[i] * s1 for i in range(DP // 16)]
            ms[6] = ms[6] + cntv
            for i in range(DP // 16):
                msg[j, pl.ds(i * 16, 16)] = ms[i]
            return carry
        lax.fori_loop(0, K, _ebody, 0)
        pltpu.sync_copy(msg, acc.at[db], add=True)

    sets = [(sb0, db0, tb0, nb0, semi0), (sb1, db1, tb1, nb1, semi1),
            (sb2, db2, tb2, nb2, semi2), (sb3, db3, tb3, nb3, semi3)]
    rowss = [(rows0, sem0), (rows1, sem1)]

    idx_start(0, *sets[0])
    idx_wait(0, *sets[0])
    gather_start(sets[0][0], *rowss[0])
    idx_start(1, *sets[1])

    def outer(i, carry):
        for b in range(4):
            g = 4 * i + b
            nxt2 = 4 * i + b + 2

            def start2():
                idx_start(nxt2, *sets[(b + 2) % 4])
            if b < 2:
                start2()
            else:
                pl.when(nxt2 < NCHUNK)(start2)

            def advance1():
                idx_wait(g + 1, *sets[(b + 1) % 4])
                gather_start(sets[(b + 1) % 4][0], *rowss[(b + 1) % 2])
            if b < 3:
                advance1()
            else:
                pl.when(g + 1 < NCHUNK)(advance1)

            sbx, dbx, tbx, nbx, _ = sets[b]
            compute(sbx, dbx, tbx, nbx, *rowss[b % 2])
        return carry
    lax.fori_loop(0, NCHUNK // 4, outer, 0)

    plsc.subcore_barrier()
    pltpu.sync_copy(acc.at[pl.ds(s * RPS, RPS)], outp.at[c, pl.ds(s * RPS, RPS)])


def _make_sc_conv():
    mesh = plsc.VectorSubcoreMesh(core_axis_name="c", subcore_axis_name="s")
    return pl.kernel(
        _sc_body,
        mesh=mesh,
        compiler_params=pltpu.CompilerParams(needs_layout_passes=False),
        out_type=jax.ShapeDtypeStruct((2, NACC, DP), _f32),
        scratch_types=(
            [pltpu.VMEM((K,), _i32), pltpu.VMEM((K,), _i32),
             pltpu.VMEM((K,), _i32), pltpu.VMEM((K,), _f32)] * 4
            + [pltpu.VMEM((K, W2), _f32), pltpu.VMEM((K, W2), _f32),
               pltpu.VMEM((K, DP), _f32),
               pltpu.VMEM((16,), _f32), pltpu.VMEM((16,), _f32),
               pltpu.VMEM_SHARED((NACC, DP), _f32)]
            + [pltpu.SemaphoreType.DMA] * 6
        ),
    )


_BL = 400


def _prep_body(x_ref, basis_ref, xcat_ref):
    x = x_ref[...]
    h0 = jnp.dot(x, basis_ref[0], preferred_element_type=_f32)
    h1 = jnp.dot(x, basis_ref[1], preferred_element_type=_f32)
    z = jnp.zeros((x.shape[0], DP - D), _f32)
    xcat_ref[...] = jnp.concatenate([h0, z, h1, z], axis=1)


def _tc_prep(x, basis):
    return pl.pallas_call(
        _prep_body,
        grid=(N // _BL,),
        in_specs=[
            pl.BlockSpec((_BL, D), lambda i: (i, 0)),
            pl.BlockSpec((2, D, D), lambda i: (0, 0, 0)),
        ],
        out_specs=pl.BlockSpec((_BL, W2), lambda i: (i, 0)),
        out_shape=jax.ShapeDtypeStruct((N, W2), _f32),
    )(x, basis)


def _junction_body(relu, p_ref, x_ref, root_ref, bias_ref, basis_ref,
                   xn_ref, xcat_ref):
    pr = p_ref[0] + p_ref[1]
    cnt = jnp.maximum(pr[:, D:D + 1], 1.0)
    agg = pr[:, :D] / cnt
    h = agg + jnp.dot(x_ref[...], root_ref[...],
                      preferred_element_type=_f32) + bias_ref[...]
    if relu:
        h = jnp.maximum(h, 0.0)
    xn_ref[...] = h
    h0 = jnp.dot(h, basis_ref[0], preferred_element_type=_f32)
    h1 = jnp.dot(h, basis_ref[1], preferred_element_type=_f32)
    z = jnp.zeros((h.shape[0], DP - D), _f32)
    xcat_ref[...] = jnp.concatenate([h0, z, h1, z], axis=1)


def _tc_junction(p, x, root, bias, basis_next, relu):
    return pl.pallas_call(
        functools.partial(_junction_body, relu),
        grid=(N // _BL,),
        in_specs=[
            pl.BlockSpec((2, _BL, DP), lambda i: (0, i, 0)),
            pl.BlockSpec((_BL, D), lambda i: (i, 0)),
            pl.BlockSpec((D, D), lambda i: (0, 0)),
            pl.BlockSpec((1, D), lambda i: (0, 0)),
            pl.BlockSpec((2, D, D), lambda i: (0, 0, 0)),
        ],
        out_specs=[
            pl.BlockSpec((_BL, D), lambda i: (i, 0)),
            pl.BlockSpec((_BL, W2), lambda i: (i, 0)),
        ],
        out_shape=[
            jax.ShapeDtypeStruct((N, D), _f32),
            jax.ShapeDtypeStruct((N, W2), _f32),
        ],
    )(p, x, root, bias, basis_next)


def _final_body(p_ref, x_ref, root_ref, bias_ref, out_ref):
    pr = p_ref[0] + p_ref[1]
    cnt = jnp.maximum(pr[:, D:D + 1], 1.0)
    agg = pr[:, :D] / cnt
    out_ref[...] = agg + jnp.dot(x_ref[...], root_ref[...],
                                 preferred_element_type=_f32) + bias_ref[...]


def _tc_final(p, x, root, bias):
    return pl.pallas_call(
        _final_body,
        grid=(N // _BL,),
        in_specs=[
            pl.BlockSpec((2, _BL, DP), lambda i: (0, i, 0)),
            pl.BlockSpec((_BL, D), lambda i: (i, 0)),
            pl.BlockSpec((D, D), lambda i: (0, 0)),
            pl.BlockSpec((1, D), lambda i: (0, 0)),
        ],
        out_specs=pl.BlockSpec((_BL, D), lambda i: (i, 0)),
        out_shape=jax.ShapeDtypeStruct((N, D), _f32),
    )(p, x, root, bias)


def kernel(entity, edge_index, edge_type, edge_norm, emb,
           basis1, att1, root1, bias1,
           basis2, att2, root2, bias2,
           basis3, att3, root3, bias3):
    x0 = jnp.take(emb, entity, axis=0)

    npad = EPAD - E
    pad_src = (jnp.arange(npad, dtype=_i32) * 97) % N
    pad_dst = N + (jnp.arange(npad, dtype=_i32) % (NACC - N))
    srcs = jnp.concatenate([edge_index[0], pad_src])
    dsts = jnp.concatenate([edge_index[1], pad_dst])
    typs = jnp.concatenate([edge_type, jnp.zeros((npad,), _i32)])
    nrms = jnp.concatenate([edge_norm, jnp.zeros((npad,), _f32)])

    a0 = {1: att1[:, 0], 2: att2[:, 0], 3: att3[:, 0]}
    a1 = {1: att1[:, 1], 2: att2[:, 1], 3: att3[:, 1]}
    basis = {1: basis1, 2: basis2, 3: basis3}
    root = {1: root1, 2: root2, 3: root3}
    bias = {1: bias1.reshape(1, D), 2: bias2.reshape(1, D),
            3: bias3.reshape(1, D)}

    sc_conv = _make_sc_conv()

    def conv(xcat, w):
        return sc_conv(xcat, srcs, dsts, typs, nrms, a0[w], a1[w])

    xcat = _tc_prep(x0, basis[1])
    p = conv(xcat, 1)
    x1, xcat = _tc_junction(p, x0, root[1], bias[1], basis[1], False)
    p = conv(xcat, 1)
    xA, xcat = _tc_junction(p, x1, root[1], bias[1], basis[2], True)
    p = conv(xcat, 2)
    x2, xcat = _tc_junction(p, xA, root[2], bias[2], basis[1], False)
    p = conv(xcat, 1)
    xB, xcat = _tc_junction(p, x2, root[1], bias[1], basis[3], True)
    p = conv(xcat, 3)
    return _tc_final(p, xB, root[3], bias[3])


# speedup vs baseline: 1.3437x; 1.3437x over previous
import functools

import jax
import jax.numpy as jnp
from jax import lax
from jax.experimental import pallas as pl
from jax.experimental.pallas import tpu as pltpu
from jax.experimental.pallas import tpu_sc as plsc

N = 10000
E = 160000
D = 100
DP = 128
W2 = 2 * DP
NACC = 10112
NW = 32
K = 64
EPW = 5120
EPAD = NW * EPW
NCHUNK = EPW // K
RPS = NACC // 16

_f32 = jnp.float32
_i32 = jnp.int32


def _sc_body(xcat, srcs, dsts, typs, nrms, a0h, a1h, outp,
             sb0, db0, tb0, nb0, sb1, db1, tb1, nb1,
             sb2, db2, tb2, nb2, sb3, db3, tb3, nb3,
             rows0, rows1, msg, a0v, a1v,
             acc, sem0, sem1, semi0, semi1, semi2, semi3):
    c = lax.axis_index("c")
    s = lax.axis_index("s")
    wbase = (c * 16 + s) * EPW

    pltpu.sync_copy(a0h, a0v)
    pltpu.sync_copy(a1h, a1v)

    lanes = lax.iota(_i32, 16)
    cntv = jnp.where(lanes == 4, 1.0, 0.0).astype(_f32)
    zv = jnp.zeros((16,), _f32)

    def zb(j, carry):
        for i in range(DP // 16):
            msg[j, pl.ds(i * 16, 16)] = zv
        return carry
    lax.fori_loop(0, K, zb, 0)

    nfull = RPS // K
    for t in range(nfull):
        pltpu.sync_copy(msg, acc.at[pl.ds(s * RPS + t * K, K)])
    rem = RPS - nfull * K
    if rem:
        pltpu.sync_copy(msg.at[pl.ds(0, rem)],
                        acc.at[pl.ds(s * RPS + nfull * K, rem)])
    plsc.subcore_barrier()

    def idx_copies(g, sb, db, tb, nb, semi):
        base = wbase + g * K
        return [
            pltpu.make_async_copy(srcs.at[pl.ds(base, K)], sb, semi),
            pltpu.make_async_copy(dsts.at[pl.ds(base, K)], db, semi),
            pltpu.make_async_copy(typs.at[pl.ds(base, K)], tb, semi),
            pltpu.make_async_copy(nrms.at[pl.ds(base, K)], nb, semi),
        ]

    def idx_start(g, sb, db, tb, nb, semi):
        for cp in idx_copies(g, sb, db, tb, nb, semi):
            cp.start()

    def idx_wait(g, sb, db, tb, nb, semi):
        for cp in idx_copies(g, sb, db, tb, nb, semi):
            cp.wait()

    def gather_start(sb, rows, sem):
        pltpu.make_async_copy(xcat.at[sb], rows, sem).start()

    def compute(sb, db, tb, nb, rows, sem):
        pltpu.make_async_copy(xcat.at[sb], rows, sem).wait()

        @functools.partial(plsc.parallel_loop, 0, K, unroll=2)
        def _(j):
            jv = jnp.broadcast_to(j, (16,)).astype(_i32)
            tj = plsc.load_gather(tb, [jv])
            nj = plsc.load_gather(nb, [jv])
            s0 = plsc.load_gather(a0v, [tj]) * nj
            s1 = plsc.load_gather(a1v, [tj]) * nj
            r0s = [rows[j, pl.ds(i * 16, 16)] for i in range(DP // 16)]
            r1s = [rows[j, pl.ds(DP + i * 16, 16)] for i in range(DP // 16)]
            ms = [r0s[i] * s0 + r1s[i] * s1 for i in range(DP // 16)]
            ms[6] = ms[6] + cntv
            for i in range(DP // 16):
                msg[j, pl.ds(i * 16, 16)] = ms[i]
        pltpu.sync_copy(msg, acc.at[db], add=True)

    sets = [(sb0, db0, tb0, nb0, semi0), (sb1, db1, tb1, nb1, semi1),
            (sb2, db2, tb2, nb2, semi2), (sb3, db3, tb3, nb3, semi3)]
    rowss = [(rows0, sem0), (rows1, sem1)]

    idx_start(0, *sets[0])
    idx_wait(0, *sets[0])
    gather_start(sets[0][0], *rowss[0])
    idx_start(1, *sets[1])

    def outer(i, carry):
        for b in range(4):
            g = 4 * i + b
            nxt2 = 4 * i + b + 2

            def start2():
                idx_start(nxt2, *sets[(b + 2) % 4])
            if b < 2:
                start2()
            else:
                pl.when(nxt2 < NCHUNK)(start2)

            def advance1():
                idx_wait(g + 1, *sets[(b + 1) % 4])
                gather_start(sets[(b + 1) % 4][0], *rowss[(b + 1) % 2])
            if b < 3:
                advance1()
            else:
                pl.when(g + 1 < NCHUNK)(advance1)

            sbx, dbx, tbx, nbx, _ = sets[b]
            compute(sbx, dbx, tbx, nbx, *rowss[b % 2])
        return carry
    lax.fori_loop(0, NCHUNK // 4, outer, 0)

    plsc.subcore_barrier()
    pltpu.sync_copy(acc.at[pl.ds(s * RPS, RPS)], outp.at[c, pl.ds(s * RPS, RPS)])


def _make_sc_conv():
    mesh = plsc.VectorSubcoreMesh(core_axis_name="c", subcore_axis_name="s")
    return pl.kernel(
        _sc_body,
        mesh=mesh,
        compiler_params=pltpu.CompilerParams(needs_layout_passes=False),
        out_type=jax.ShapeDtypeStruct((2, NACC, DP), _f32),
        scratch_types=(
            [pltpu.VMEM((K,), _i32), pltpu.VMEM((K,), _i32),
             pltpu.VMEM((K,), _i32), pltpu.VMEM((K,), _f32)] * 4
            + [pltpu.VMEM((K, W2), _f32), pltpu.VMEM((K, W2), _f32),
               pltpu.VMEM((K, DP), _f32),
               pltpu.VMEM((16,), _f32), pltpu.VMEM((16,), _f32),
               pltpu.VMEM_SHARED((NACC, DP), _f32)]
            + [pltpu.SemaphoreType.DMA] * 6
        ),
    )


_BL = 400


def _prep_body(x_ref, basis_ref, xcat_ref):
    x = x_ref[...]
    h0 = jnp.dot(x, basis_ref[0], preferred_element_type=_f32)
    h1 = jnp.dot(x, basis_ref[1], preferred_element_type=_f32)
    z = jnp.zeros((x.shape[0], DP - D), _f32)
    xcat_ref[...] = jnp.concatenate([h0, z, h1, z], axis=1)


def _tc_prep(x, basis):
    return pl.pallas_call(
        _prep_body,
        grid=(N // _BL,),
        in_specs=[
            pl.BlockSpec((_BL, D), lambda i: (i, 0)),
            pl.BlockSpec((2, D, D), lambda i: (0, 0, 0)),
        ],
        out_specs=pl.BlockSpec((_BL, W2), lambda i: (i, 0)),
        out_shape=jax.ShapeDtypeStruct((N, W2), _f32),
    )(x, basis)


def _junction_body(relu, p_ref, x_ref, root_ref, bias_ref, basis_ref,
                   xn_ref, xcat_ref):
    pr = p_ref[0] + p_ref[1]
    cnt = jnp.maximum(pr[:, D:D + 1], 1.0)
    agg = pr[:, :D] / cnt
    h = agg + jnp.dot(x_ref[...], root_ref[...],
                      preferred_element_type=_f32) + bias_ref[...]
    if relu:
        h = jnp.maximum(h, 0.0)
    xn_ref[...] = h
    h0 = jnp.dot(h, basis_ref[0], preferred_element_type=_f32)
    h1 = jnp.dot(h, basis_ref[1], preferred_element_type=_f32)
    z = jnp.zeros((h.shape[0], DP - D), _f32)
    xcat_ref[...] = jnp.concatenate([h0, z, h1, z], axis=1)


def _tc_junction(p, x, root, bias, basis_next, relu):
    return pl.pallas_call(
        functools.partial(_junction_body, relu),
        grid=(N // _BL,),
        in_specs=[
            pl.BlockSpec((2, _BL, DP), lambda i: (0, i, 0)),
            pl.BlockSpec((_BL, D), lambda i: (i, 0)),
            pl.BlockSpec((D, D), lambda i: (0, 0)),
            pl.BlockSpec((1, D), lambda i: (0, 0)),
            pl.BlockSpec((2, D, D), lambda i: (0, 0, 0)),
        ],
        out_specs=[
            pl.BlockSpec((_BL, D), lambda i: (i, 0)),
            pl.BlockSpec((_BL, W2), lambda i: (i, 0)),
        ],
        out_shape=[
            jax.ShapeDtypeStruct((N, D), _f32),
            jax.ShapeDtypeStruct((N, W2), _f32),
        ],
    )(p, x, root, bias, basis_next)


def _final_body(p_ref, x_ref, root_ref, bias_ref, out_ref):
    pr = p_ref[0] + p_ref[1]
    cnt = jnp.maximum(pr[:, D:D + 1], 1.0)
    agg = pr[:, :D] / cnt
    out_ref[...] = agg + jnp.dot(x_ref[...], root_ref[...],
                                 preferred_element_type=_f32) + bias_ref[...]


def _tc_final(p, x, root, bias):
    return pl.pallas_call(
        _final_body,
        grid=(N // _BL,),
        in_specs=[
            pl.BlockSpec((2, _BL, DP), lambda i: (0, i, 0)),
            pl.BlockSpec((_BL, D), lambda i: (i, 0)),
            pl.BlockSpec((D, D), lambda i: (0, 0)),
            pl.BlockSpec((1, D), lambda i: (0, 0)),
        ],
        out_specs=pl.BlockSpec((_BL, D), lambda i: (i, 0)),
        out_shape=jax.ShapeDtypeStruct((N, D), _f32),
    )(p, x, root, bias)


def kernel(entity, edge_index, edge_type, edge_norm, emb,
           basis1, att1, root1, bias1,
           basis2, att2, root2, bias2,
           basis3, att3, root3, bias3):
    x0 = jnp.take(emb, entity, axis=0)

    npad = EPAD - E
    pad_src = (jnp.arange(npad, dtype=_i32) * 97) % N
    pad_dst = N + (jnp.arange(npad, dtype=_i32) % (NACC - N))
    srcs = jnp.concatenate([edge_index[0], pad_src])
    dsts = jnp.concatenate([edge_index[1], pad_dst])
    typs = jnp.concatenate([edge_type, jnp.zeros((npad,), _i32)])
    nrms = jnp.concatenate([edge_norm, jnp.zeros((npad,), _f32)])

    a0 = {1: att1[:, 0], 2: att2[:, 0], 3: att3[:, 0]}
    a1 = {1: att1[:, 1], 2: att2[:, 1], 3: att3[:, 1]}
    basis = {1: basis1, 2: basis2, 3: basis3}
    root = {1: root1, 2: root2, 3: root3}
    bias = {1: bias1.reshape(1, D), 2: bias2.reshape(1, D),
            3: bias3.reshape(1, D)}

    sc_conv = _make_sc_conv()

    def conv(xcat, w):
        return sc_conv(xcat, srcs, dsts, typs, nrms, a0[w], a1[w])

    xcat = _tc_prep(x0, basis[1])
    p = conv(xcat, 1)
    x1, xcat = _tc_junction(p, x0, root[1], bias[1], basis[1], False)
    p = conv(xcat, 1)
    xA, xcat = _tc_junction(p, x1, root[1], bias[1], basis[2], True)
    p = conv(xcat, 2)
    x2, xcat = _tc_junction(p, xA, root[2], bias[2], basis[1], False)
    p = conv(xcat, 1)
    xB, xcat = _tc_junction(p, x2, root[1], bias[1], basis[3], True)
    p = conv(xcat, 3)
    return _tc_final(p, xB, root[3], bias[3])
